# baseline (speedup 1.0000x reference)
import functools

import jax
import jax.numpy as jnp
from jax.experimental import pallas as pl
from jax.experimental.pallas import tpu as pltpu

_K = 20
_KSEL = _K + 1


def _extraction_mask(R, N, iota, d):
    inf = jnp.float32(jnp.inf)

    def extract(i, carry):
        dd, mask = carry
        mv = jnp.min(dd, axis=1, keepdims=True)
        eq = dd == mv
        jm = jnp.min(jnp.where(eq, iota, N), axis=1, keepdims=True)
        sel = iota == jm
        dd = jnp.where(sel, inf, dd)
        keep = jnp.logical_and(sel, i > 0)
        mask = mask + jnp.where(keep, 1.0, 0.0)
        return dd, mask

    mask0 = jnp.zeros((R, N), jnp.float32)
    _, mask = jax.lax.fori_loop(0, _KSEL, extract, (d, mask0))
    return mask


def _smoothness_body(R, N, pts_r_ref, ptsT_ref, emb_r_ref, embT_ref,
                     out_ref, sq_ref, se_ref):
    b = pl.program_id(0)
    rb = pl.program_id(1)

    @pl.when(rb == 0)
    def _():
        ptsT = ptsT_ref[0]
        sq_ref[...] = jnp.sum(ptsT * ptsT, axis=0, keepdims=True)
        eT = embT_ref[0]
        se_ref[...] = jnp.sum(eT * eT, axis=0, keepdims=True)

    pr = pts_r_ref[0]
    er = emb_r_ref[0]
    sqr = jnp.sum(pr * pr, axis=1, keepdims=True)
    ser = jnp.sum(er * er, axis=1, keepdims=True)

    pdot = jnp.dot(pr, ptsT_ref[0], preferred_element_type=jnp.float32)
    d = jnp.maximum(sq_ref[...] + sqr - 2.0 * pdot, 0.0)
    edot = jnp.dot(er, embT_ref[0], preferred_element_type=jnp.float32)

    iota = jax.lax.broadcasted_iota(jnp.int32, (R, N), 1)

    dbits = jax.lax.bitcast_convert_type(d, jnp.int32)
    C = N // 128
    cmin = jnp.min(dbits.reshape(R, C, 128), axis=2)
    lo0 = jnp.min(dbits, axis=1, keepdims=True)

    def cstep(_, carry):
        lo, hi = carry
        mid = lo + jax.lax.shift_right_logical(hi - lo, 1)
        cnt = jnp.sum((cmin <= mid).astype(jnp.int32), axis=1,
                      keepdims=True)
        ge = cnt >= _KSEL
        return jnp.where(ge, lo, mid + 1), jnp.where(ge, mid, hi)

    _, hi0 = jax.lax.fori_loop(
        0, 31, cstep, (lo0, jnp.full((R, 1), 0x7F800000, jnp.int32)))

    def bcond(carry):
        lo, hi = carry
        return jnp.any(lo < hi)

    def bstep(carry):
        lo, hi = carry
        mid = lo + jax.lax.shift_right_logical(hi - lo, 1)
        le = dbits <= mid
        cnt = jnp.sum(le.astype(jnp.int32), axis=1, keepdims=True)
        ge = cnt >= _KSEL
        hi = jnp.where(ge, mid, hi)
        lo = jnp.where(ge, lo, mid + 1)
        return lo, hi

    v21, _ = jax.lax.while_loop(bcond, bstep, (lo0, hi0))

    lt = dbits < v21
    c_lt = jnp.sum(lt.astype(jnp.int32), axis=1, keepdims=True)
    need = _KSEL - c_lt
    fastok = jnp.all(need == 1)

    def fast_mask():
        eq = dbits == v21
        jm = jnp.min(jnp.where(eq, iota, N), axis=1, keepdims=True)
        tie = (iota == jm).astype(jnp.float32)
        dminb = jnp.min(dbits, axis=1, keepdims=True)
        eqm = dbits == dminb
        jmm = jnp.min(jnp.where(eqm, iota, N), axis=1, keepdims=True)
        drop = (iota == jmm).astype(jnp.float32)
        return lt.astype(jnp.float32) + tie - drop

    mask = jax.lax.cond(fastok, fast_mask,
                        lambda: _extraction_mask(R, N, iota, d))

    e2 = jnp.maximum(ser + se_ref[...] - 2.0 * edot, 0.0)
    part = jnp.sum(mask * jnp.sqrt(e2))

    @pl.when(jnp.logical_and(b == 0, rb == 0))
    def _():
        out_ref[...] = jnp.zeros((1, 1), jnp.float32)

    out_ref[...] += jnp.full((1, 1), part, jnp.float32)


def _smoothness_sum(points_pad, pointsT, embeddings, embeddingsT, R):
    B, N, _ = points_pad.shape
    D = embeddings.shape[2]
    body = functools.partial(_smoothness_body, R, N)
    out = pl.pallas_call(
        body,
        grid=(B, N // R),
        in_specs=[
            pl.BlockSpec((1, R, 8), lambda b, r: (b, r, 0)),
            pl.BlockSpec((1, 8, N), lambda b, r: (b, 0, 0)),
            pl.BlockSpec((1, R, D), lambda b, r: (b, r, 0)),
            pl.BlockSpec((1, D, N), lambda b, r: (b, 0, 0)),
        ],
        out_specs=pl.BlockSpec((1, 1), lambda b, r: (0, 0)),
        out_shape=jax.ShapeDtypeStruct((1, 1), jnp.float32),
        scratch_shapes=[
            pltpu.VMEM((1, N), jnp.float32),
            pltpu.VMEM((1, N), jnp.float32),
        ],
    )(points_pad, pointsT, embeddings, embeddingsT)
    return out[0, 0]


@jax.jit
def kernel(points, embeddings):
    B, N, _ = points.shape
    points_pad = jnp.pad(points, ((0, 0), (0, 0), (0, 5)))
    pointsT = jnp.transpose(points_pad, (0, 2, 1))
    embeddingsT = jnp.transpose(embeddings, (0, 2, 1))
    R = 128 if N % 128 == 0 else 8
    total = _smoothness_sum(points_pad, pointsT, embeddings, embeddingsT, R)
    return total / jnp.float32(B * N * _K)


# speedup vs baseline: 2.9698x; 2.9698x over previous
import functools

import jax
import jax.numpy as jnp
from jax.experimental import pallas as pl
from jax.experimental.pallas import tpu as pltpu

_K = 20
_KSEL = _K + 1


def _extraction_mask(R, N, iota, d):
    inf = jnp.float32(jnp.inf)

    def extract(i, carry):
        dd, mask = carry
        mv = jnp.min(dd, axis=1, keepdims=True)
        eq = dd == mv
        jm = jnp.min(jnp.where(eq, iota, N), axis=1, keepdims=True)
        sel = iota == jm
        dd = jnp.where(sel, inf, dd)
        keep = jnp.logical_and(sel, i > 0)
        mask = mask + jnp.where(keep, 1.0, 0.0)
        return dd, mask

    mask0 = jnp.zeros((R, N), jnp.float32)
    _, mask = jax.lax.fori_loop(0, _KSEL, extract, (d, mask0))
    return mask


def _smoothness_body(R, N, pts_r_ref, ptsT_ref, emb_r_ref, embT_ref,
                     out_ref, sq_ref, se_ref):
    b = pl.program_id(0)
    rb = pl.program_id(1)

    @pl.when(rb == 0)
    def _():
        ptsT = ptsT_ref[0]
        sq_ref[...] = jnp.sum(ptsT * ptsT, axis=0, keepdims=True)
        eT = embT_ref[0]
        se_ref[...] = jnp.sum(eT * eT, axis=0, keepdims=True)

    pr = pts_r_ref[0]
    er = emb_r_ref[0]
    sqr = jnp.sum(pr * pr, axis=1, keepdims=True)
    ser = jnp.sum(er * er, axis=1, keepdims=True)

    pdot = jnp.dot(pr, ptsT_ref[0], preferred_element_type=jnp.float32)
    d = jnp.maximum(sq_ref[...] + sqr - 2.0 * pdot, 0.0)
    edot = jnp.dot(er, embT_ref[0], preferred_element_type=jnp.float32)

    iota = jax.lax.broadcasted_iota(jnp.int32, (R, N), 1)

    dbits = jax.lax.bitcast_convert_type(d, jnp.int32)
    lo0 = jnp.zeros((R, 1), jnp.int32)
    hi0 = jnp.full((R, 1), 0x7F800000, jnp.int32)

    def bstep(_, carry):
        lo, hi, csel = carry
        mid = lo + jax.lax.shift_right_logical(hi - lo, 1)
        acc = jnp.zeros((R, 128), jnp.int32)
        for w in range(N // 128):
            blk = jax.lax.slice(dbits, (0, w * 128), (R, (w + 1) * 128))
            acc = acc + jnp.where(blk <= mid, 1, 0)
        cnt = jnp.sum(acc, axis=1, keepdims=True)
        ge = cnt >= _KSEL
        hi = jnp.where(ge, mid, hi)
        lo = jnp.where(ge, lo, mid + 1)
        csel = jnp.where(ge, cnt, csel)
        return lo, hi, csel

    v21, _, c_le = jax.lax.fori_loop(
        0, 31, bstep, (lo0, hi0, jnp.zeros((R, 1), jnp.int32)))

    fastok = jnp.all(c_le == _KSEL)

    en = jnp.sqrt(jnp.maximum(ser + se_ref[...] - 2.0 * edot, 0.0))

    def fast_part():
        le = dbits <= v21
        diag = iota == (jax.lax.broadcasted_iota(jnp.int32, (R, N), 0)
                        + rb * R)
        keep = jnp.logical_and(le, jnp.logical_not(diag))
        return jnp.sum(jnp.where(keep, en, 0.0))

    def slow_part():
        return jnp.sum(_extraction_mask(R, N, iota, d) * en)

    part = jax.lax.cond(fastok, fast_part, slow_part)

    @pl.when(jnp.logical_and(b == 0, rb == 0))
    def _():
        out_ref[...] = jnp.zeros((1, 1), jnp.float32)

    out_ref[...] += jnp.full((1, 1), part, jnp.float32)


def _smoothness_sum(points_pad, pointsT, embeddings, embeddingsT, R):
    B, N, _ = points_pad.shape
    D = embeddings.shape[2]
    body = functools.partial(_smoothness_body, R, N)
    out = pl.pallas_call(
        body,
        grid=(B, N // R),
        in_specs=[
            pl.BlockSpec((1, R, 8), lambda b, r: (b, r, 0)),
            pl.BlockSpec((1, 8, N), lambda b, r: (b, 0, 0)),
            pl.BlockSpec((1, R, D), lambda b, r: (b, r, 0)),
            pl.BlockSpec((1, D, N), lambda b, r: (b, 0, 0)),
        ],
        out_specs=pl.BlockSpec((1, 1), lambda b, r: (0, 0)),
        out_shape=jax.ShapeDtypeStruct((1, 1), jnp.float32),
        scratch_shapes=[
            pltpu.VMEM((1, N), jnp.float32),
            pltpu.VMEM((1, N), jnp.float32),
        ],
    )(points_pad, pointsT, embeddings, embeddingsT)
    return out[0, 0]


@jax.jit
def kernel(points, embeddings):
    B, N, _ = points.shape
    points_pad = jnp.pad(points, ((0, 0), (0, 0), (0, 5)))
    pointsT = jnp.transpose(points_pad, (0, 2, 1))
    embeddingsT = jnp.transpose(embeddings, (0, 2, 1))
    R = 128 if N % 128 == 0 else 8
    total = _smoothness_sum(points_pad, pointsT, embeddings, embeddingsT, R)
    return total / jnp.float32(B * N * _K)
